# baseline (speedup 1.0000x reference)
import functools

import jax
import jax.numpy as jnp
from jax import lax
from jax.experimental import pallas as pl
from jax.experimental.pallas import tpu as pltpu
from jax.experimental.pallas import tpu_sc as plsc

B = 16384
D = 64
L = 16
NC = 2
NS = 16
NW = NC * NS
BPW = B // NW
CHUNK = 128
NCHUNK = BPW // CHUNK
NGRP = BPW // L


def _sc_body(hidx_hbm, ridx_hbm, tidx_hbm, ent_hbm, rel_hbm, out_hbm,
             hidx_v, ridx_v, tidx_v, hrows, rrows, trows,
             out_v, sem):
    wid = lax.axis_index("s") * NC + lax.axis_index("c")
    base = wid * BPW

    for j in range(NCHUNK):
        sl = pl.ds(base + j * CHUNK, CHUNK)
        pltpu.sync_copy(hidx_hbm.at[sl], hidx_v.at[j])
        pltpu.sync_copy(ridx_hbm.at[sl], ridx_v.at[j])
        pltpu.sync_copy(tidx_hbm.at[sl], tidx_v.at[j])

    copies = []
    for j in range(NCHUNK):
        sl = pl.ds(j * CHUNK, CHUNK)
        copies.append(pltpu.async_copy(ent_hbm.at[hidx_v.at[j]], hrows.at[sl], sem))
        copies.append(pltpu.async_copy(rel_hbm.at[ridx_v.at[j]], rrows.at[sl], sem))
        copies.append(pltpu.async_copy(ent_hbm.at[tidx_v.at[j]], trows.at[sl], sem))
    for c in copies:
        c.wait()

    lane = lax.iota(jnp.int32, L)
    def body1(grp, carry):
        t0 = grp * L
        scores = jnp.zeros((L,), jnp.float32)
        for j in range(L):
            t = t0 + j
            acc = hrows[t, pl.ds(0, L)] * rrows[t, pl.ds(0, L)] * trows[t, pl.ds(0, L)]
            for g in range(1, D // L):
                sl = pl.ds(g * L, L)
                acc = acc + hrows[t, sl] * rrows[t, sl] * trows[t, sl]
            scores = jnp.where(lane == j, jnp.sum(acc), scores)
        out_v[pl.ds(t0, L)] = scores
        return carry
    lax.fori_loop(0, NGRP, body1, 0)

    pltpu.sync_copy(out_v, out_hbm.at[pl.ds(base, BPW)])


@jax.jit
def _sc_score(head_indices, rel_indices, tail_indices, entity_embedding, relation_embedding):
    run = functools.partial(
        pl.kernel,
        mesh=plsc.VectorSubcoreMesh(core_axis_name="c", subcore_axis_name="s"),
        compiler_params=pltpu.CompilerParams(
            needs_layout_passes=False, use_tc_tiling_on_sc=False),
        out_type=jax.ShapeDtypeStruct((B,), jnp.float32),
        scratch_types=[
            pltpu.VMEM((NCHUNK, CHUNK), jnp.int32),
            pltpu.VMEM((NCHUNK, CHUNK), jnp.int32),
            pltpu.VMEM((NCHUNK, CHUNK), jnp.int32),
            pltpu.VMEM((BPW, D), jnp.float32),
            pltpu.VMEM((BPW, D), jnp.float32),
            pltpu.VMEM((BPW, D), jnp.float32),
            pltpu.VMEM((BPW,), jnp.float32),
            pltpu.SemaphoreType.DMA,
        ],
    )(_sc_body)
    return run(head_indices, rel_indices, tail_indices,
               entity_embedding, relation_embedding)


def kernel(head_indices, rel_indices, tail_indices, entity_embedding, relation_embedding):
    scores = _sc_score(head_indices, rel_indices, tail_indices,
                       entity_embedding, relation_embedding)
    return scores.reshape(B, 1)
